# Initial kernel scaffold; baseline (speedup 1.0000x reference)
import functools

import jax
import jax.numpy as jnp
from jax import lax
from jax.experimental import pallas as pl
from jax.experimental.pallas import tpu as pltpu
from jax.experimental.pallas import tpu_sc as plsc

NU = 20000
NI = 30000
NN = NU + NI
NP = 50176
D = 64
HD = 32
E = 800000
B = 1024
K = 1000
TEMP = 0.1
NTILES = 16
ROWS_PT = NP // NTILES
EDGES_PT = E // NTILES
W = 80
NCH = EDGES_PT // W
ZR = 224
NZ = ROWS_PT // ZR
HR = 112
HC = 448
CH = 1568
NSTEP = NP // CH

_MESH = plsc.VectorSubcoreMesh(core_axis_name="c", subcore_axis_name="s")
f32 = jnp.float32
i32 = jnp.int32


def _deg_body(ei, deg, hist, idx, rowidx, acc):
    c = lax.axis_index("c")
    s = lax.axis_index("s")
    pltpu.sync_copy(ei.at[c, pl.ds(s * EDGES_PT, EDGES_PT)], idx)

    zero16 = jnp.zeros((16,), f32)
    iota16 = lax.iota(i32, 16)

    def zb(k, _):
        hist[k // (HC // 16), pl.ds((k % (HC // 16)) * 16, 16)] = zero16
        return 0

    lax.fori_loop(0, HR * (HC // 16), zb, 0)

    for t in range(HR // 16):
        rowidx[pl.ds(t * 16, 16)] = iota16 + t * 16

    pltpu.sync_copy(hist.at[pl.ds(0, HR // NTILES)],
                    acc.at[pl.ds(s * (HR // NTILES), HR // NTILES)])
    plsc.subcore_barrier()

    def hb(k, _):
        v = idx[pl.ds(k * 16, 16)]
        cnt, last = plsc.scan_count(v)
        plsc.addupdate_scatter(hist, [v // HC, v % HC],
                               cnt.astype(f32), mask=last)
        return 0

    lax.fori_loop(0, EDGES_PT // 16, hb, 0)

    pltpu.sync_copy(hist, acc.at[rowidx], add=True)
    plsc.subcore_barrier()
    pltpu.sync_copy(acc.at[pl.ds(s * (HR // NTILES), HR // NTILES)],
                    deg.at[c, pl.ds(s * (HR // NTILES), HR // NTILES)])


@jax.jit
def _deg_call(ei):
    return pl.kernel(
        _deg_body,
        out_type=jax.ShapeDtypeStruct((2, HR, HC), f32),
        mesh=_MESH,
        scratch_types=[
            pltpu.VMEM((HR, HC), f32),
            pltpu.VMEM((EDGES_PT,), i32),
            pltpu.VMEM((HR,), i32),
            pltpu.VMEM_SHARED((HR, HC), f32),
        ],
    )(ei)


def _layer_body(hs, src3, dst3, agg, srcb, dstb, rows0, rows1, zbuf,
                semg0, semg1, sems0, sems1, acc):
    c = lax.axis_index("c")
    s = lax.axis_index("s")
    pltpu.sync_copy(src3.at[s], srcb)
    pltpu.sync_copy(dst3.at[s], dstb)

    zero16 = jnp.zeros((16,), f32)

    def zb(k, _):
        zbuf[k // 2, pl.ds((k % 2) * 16, 16)] = zero16
        return 0

    lax.fori_loop(0, ZR * 2, zb, 0)
    for t in range(NZ):
        pltpu.sync_copy(zbuf, acc.at[pl.ds(s * ROWS_PT + t * ZR, ZR)])
    plsc.subcore_barrier()

    tbl = hs.at[c]

    def gather(k, buf, sem):
        pltpu.async_copy(tbl.at[srcb.at[k]], buf, sem)

    def scat(k, buf, sem):
        pltpu.async_copy(buf, acc.at[dstb.at[k]], sem, add=True)

    def wait_g(buf, sem):
        pltpu.make_async_copy(tbl.at[srcb.at[0]], buf, sem).wait()

    def wait_s(buf, sem):
        pltpu.make_async_copy(buf, acc.at[dstb.at[0]], sem).wait()

    gather(0, rows0, semg0)

    def body(j, _):
        k0 = 2 * j

        @pl.when(j > 0)
        def _():
            wait_s(rows1, sems1)

        gather(k0 + 1, rows1, semg1)
        wait_g(rows0, semg0)
        scat(k0, rows0, sems0)

        wait_s(rows0, sems0)

        @pl.when(j < NCH // 2 - 1)
        def _():
            gather(k0 + 2, rows0, semg0)

        wait_g(rows1, semg1)
        scat(k0 + 1, rows1, sems1)
        return 0

    lax.fori_loop(0, NCH // 2, body, 0)
    wait_s(rows1, sems1)
    gather(NCH - 1, rows0, semg0)
    wait_g(rows0, semg0)
    scat(NCH - 1, rows0, sems0)
    wait_s(rows0, sems0)
    plsc.subcore_barrier()

    for t in range(NZ):
        r0 = s * ROWS_PT + t * ZR
        pltpu.sync_copy(acc.at[pl.ds(r0, ZR)], agg.at[c, pl.ds(r0, ZR)])


@jax.jit
def _layer_call(hs, src3, dst3):
    return pl.kernel(
        _layer_body,
        out_type=jax.ShapeDtypeStruct((2, NP, HD), f32),
        mesh=_MESH,
        scratch_types=[
            pltpu.VMEM((NCH, W), i32),
            pltpu.VMEM((NCH, W), i32),
            pltpu.VMEM((W, HD), f32),
            pltpu.VMEM((W, HD), f32),
            pltpu.VMEM((ZR, HD), f32),
            pltpu.SemaphoreType.DMA,
            pltpu.SemaphoreType.DMA,
            pltpu.SemaphoreType.DMA,
            pltpu.SemaphoreType.DMA,
            pltpu.VMEM_SHARED((NP, HD), f32),
        ],
    )(hs, src3, dst3)


def _gather_body(all0s, hs1, hs2, agg3, do_h, di_h, ids_h, user_h, pos_h,
                 u2c_h, i2c_h, ucent_h, icent_h,
                 G0, G1, G2, G3, dog, dig, UC, IC,
                 idsb, rowsb, svalb, sidb, clb, crows, sem):
    c = lax.axis_index("c")
    s = lax.axis_index("s")

    for tbl, out in ((all0s, G0), (hs1, G1), (hs2, G2), (agg3, G3)):
        for j in range(2):
            base = s * 192 + j * 96
            pltpu.sync_copy(ids_h.at[pl.ds(base, 96)], idsb)
            pltpu.async_copy(tbl.at[c].at[idsb], rowsb, sem).wait()
            pltpu.sync_copy(rowsb, out.at[c, pl.ds(base, 96)])

    @pl.when(c == 0)
    def _():
        for j in range(2):
            base = s * 192 + j * 96
            pltpu.sync_copy(ids_h.at[pl.ds(base, 96)], idsb)
            pltpu.async_copy(do_h.at[idsb], svalb, sem).wait()
            pltpu.sync_copy(svalb, dog.at[pl.ds(base, 96)])
        cb = s * 64
        pltpu.sync_copy(user_h.at[pl.ds(cb, 64)], sidb)
        pltpu.async_copy(u2c_h.at[sidb], clb, sem).wait()
        pltpu.async_copy(ucent_h.at[clb], crows, sem).wait()
        pltpu.sync_copy(crows, UC.at[pl.ds(cb, 64)])

    @pl.when(c == 1)
    def _():
        for j in range(2):
            base = s * 192 + j * 96
            pltpu.sync_copy(ids_h.at[pl.ds(base, 96)], idsb)
            pltpu.async_copy(di_h.at[idsb], svalb, sem).wait()
            pltpu.sync_copy(svalb, dig.at[pl.ds(base, 96)])
        cb = s * 64
        pltpu.sync_copy(pos_h.at[pl.ds(cb, 64)], sidb)
        pltpu.async_copy(i2c_h.at[sidb], clb, sem).wait()
        pltpu.async_copy(icent_h.at[clb], crows, sem).wait()
        pltpu.sync_copy(crows, IC.at[pl.ds(cb, 64)])


@jax.jit
def _gather_call(all0s, hs1, hs2, agg3, do_h, di_h, ids_all, user,
                 pos_item, u2c, i2c, ucent, icent):
    return pl.kernel(
        _gather_body,
        out_type=(
            jax.ShapeDtypeStruct((2, 3 * B, HD), f32),
            jax.ShapeDtypeStruct((2, 3 * B, HD), f32),
            jax.ShapeDtypeStruct((2, 3 * B, HD), f32),
            jax.ShapeDtypeStruct((2, 3 * B, HD), f32),
            jax.ShapeDtypeStruct((3 * B, 1), f32),
            jax.ShapeDtypeStruct((3 * B, 1), f32),
            jax.ShapeDtypeStruct((B, D), f32),
            jax.ShapeDtypeStruct((B, D), f32),
        ),
        mesh=_MESH,
        scratch_types=[
            pltpu.VMEM((96,), i32),
            pltpu.VMEM((96, HD), f32),
            pltpu.VMEM((96, 1), f32),
            pltpu.VMEM((64,), i32),
            pltpu.VMEM((64,), i32),
            pltpu.VMEM((64, D), f32),
            pltpu.SemaphoreType.DMA,
        ],
    )(all0s, hs1, hs2, agg3, do_h, di_h, ids_all, user, pos_item,
      u2c, i2c, ucent, icent)


RB = 6272


def _prep_body(all0s_ref, deg_ref, hs0_ref, do_ref, di_ref, dido_ref):
    dego = deg_ref[0]
    degi = deg_ref[1]
    do = lax.rsqrt(jnp.where(dego > 0, dego, 1.0))
    di = lax.rsqrt(jnp.where(degi > 0, degi, 1.0))
    do_ref[...] = do
    di_ref[...] = di
    dido_ref[...] = do * di
    hs0_ref[...] = all0s_ref[...] * do[None]


@jax.jit
def _prep_call(all0s, deg3):
    return pl.pallas_call(
        _prep_body,
        grid=(NP // RB,),
        in_specs=[
            pl.BlockSpec((2, RB, HD), lambda i: (0, i, 0)),
            pl.BlockSpec((2, RB, 1), lambda i: (0, i, 0)),
        ],
        out_specs=[
            pl.BlockSpec((2, RB, HD), lambda i: (0, i, 0)),
            pl.BlockSpec((RB, 1), lambda i: (i, 0)),
            pl.BlockSpec((RB, 1), lambda i: (i, 0)),
            pl.BlockSpec((RB, 1), lambda i: (i, 0)),
        ],
        out_shape=[
            jax.ShapeDtypeStruct((2, NP, HD), f32),
            jax.ShapeDtypeStruct((NP, 1), f32),
            jax.ShapeDtypeStruct((NP, 1), f32),
            jax.ShapeDtypeStruct((NP, 1), f32),
        ],
    )(all0s, deg3)


def _scale_body(agg_ref, dido_ref, hs_ref):
    hs_ref[...] = agg_ref[...] * dido_ref[...][None]


@jax.jit
def _scale_call(agg, dido):
    return pl.pallas_call(
        _scale_body,
        grid=(NP // RB,),
        in_specs=[
            pl.BlockSpec((2, RB, HD), lambda i: (0, i, 0)),
            pl.BlockSpec((RB, 1), lambda i: (i, 0)),
        ],
        out_specs=pl.BlockSpec((2, RB, HD), lambda i: (0, i, 0)),
        out_shape=jax.ShapeDtypeStruct((2, NP, HD), f32),
    )(agg, dido)


def _loss_body(all0p_ref, G0r, G1r, G2r, G3r, dogr, digr, UCr, ICr,
               ucr, icr, out_ref, n1_s, ttl_s):
    i = pl.program_id(0)
    invT = 1.0 / TEMP

    def l2n(v):
        nrm = jnp.sqrt(jnp.sum(v * v, axis=1, keepdims=True))
        return v / jnp.maximum(nrm, 1e-12)

    @pl.when(i == 0)
    def _():
        g2 = G2r[...]
        n1_s[...] = l2n(g2[0:2 * B])
        ttl_s[...] = jnp.zeros((2 * B, 1), f32)

    x = all0p_ref[...]
    xn = l2n(x)
    lg = lax.dot_general(n1_s[...], xn, (((1,), (1,)), ((), ())),
                         preferred_element_type=f32)
    col = i * CH + lax.broadcasted_iota(i32, (1, CH), 1)
    row = lax.broadcasted_iota(i32, (2 * B, 1), 0)
    m = jnp.where(row < B, col < NU, (col >= NU) & (col < NN))
    e = jnp.exp(lg * invT) * m.astype(f32)
    ttl_s[...] += jnp.sum(e, axis=1, keepdims=True)

    @pl.when(i == NSTEP - 1)
    def _():
        g0 = G0r[...]
        inv_do = 1.0 / dogr[...]
        h1 = G1r[...] * inv_do
        h2 = G2r[...] * inv_do
        h3 = G3r[...] * digr[...]
        lgall = (g0 + h1 + h2 + h3) * 0.25
        ue = lgall[0:B]
        pe = lgall[B:2 * B]
        ne = lgall[2 * B:3 * B]
        pos_s = jnp.sum(ue * pe, axis=1)
        neg_s = jnp.sum(ue * ne, axis=1)
        sig = 1.0 / (1.0 + jnp.exp(neg_s - pos_s))
        mf = -jnp.mean(jnp.log(1e-10 + sig))
        g0u = g0[0:B]
        g0p = g0[B:2 * B]
        g0n = g0[2 * B:3 * B]
        reg = (jnp.sqrt(jnp.sum(g0u * g0u)) + jnp.sqrt(jnp.sum(g0p * g0p))
               + jnp.sqrt(jnp.sum(g0n * g0n))) / B
        n1 = n1_s[...]
        nu2 = l2n(g0u)
        ni2 = l2n(g0p)
        n2 = jnp.concatenate([nu2, ni2], axis=0)
        pos_ = jnp.exp(jnp.sum(n1 * n2, axis=1, keepdims=True) * invT)
        ssl_total = -jnp.sum(jnp.log(pos_ / ttl_s[...]))
        uc = UCr[...]
        ic = ICr[...]
        pos_su = jnp.exp(jnp.sum(nu2 * uc, axis=1) * invT)
        lsu = lax.dot_general(nu2, ucr[...], (((1,), (1,)), ((), ())),
                              preferred_element_type=f32)
        ttl_su = jnp.sum(jnp.exp(lsu * invT), axis=1)
        proto_u = -jnp.sum(jnp.log(pos_su / ttl_su))
        pos_si = jnp.exp(jnp.sum(ni2 * ic, axis=1) * invT)
        lsi = lax.dot_general(ni2, icr[...], (((1,), (1,)), ((), ())),
                              preferred_element_type=f32)
        ttl_si = jnp.sum(jnp.exp(lsi * invT), axis=1)
        proto_i = -jnp.sum(jnp.log(pos_si / ttl_si))
        l0 = mf + 1e-4 * reg
        l1 = 1e-6 * ssl_total
        l2v = 8e-8 * (proto_u + proto_i)
        lane = lax.broadcasted_iota(i32, (1, 128), 1)
        out_ref[...] = jnp.where(
            lane == 0, l0, jnp.where(lane == 1, l1,
                                     jnp.where(lane == 2, l2v, 0.0)))


@jax.jit
def _loss_call(all0p, G0, G1, G2, G3, dog, dig, UC, IC, ucent, icent):
    def full(shape):
        return pl.BlockSpec(shape, lambda i, _s=shape: tuple(0 for _ in _s))
    return pl.pallas_call(
        _loss_body,
        grid=(NSTEP,),
        in_specs=[
            pl.BlockSpec((CH, D), lambda i: (i, 0)),
            full((3 * B, D)), full((3 * B, D)), full((3 * B, D)),
            full((3 * B, D)), full((3 * B, 1)), full((3 * B, 1)),
            full((B, D)), full((B, D)), full((K, D)), full((K, D)),
        ],
        out_specs=pl.BlockSpec((1, 128), lambda i: (0, 0)),
        out_shape=jax.ShapeDtypeStruct((1, 128), f32),
        scratch_shapes=[
            pltpu.VMEM((2 * B, D), f32),
            pltpu.VMEM((2 * B, 1), f32),
        ],
    )(all0p, G0, G1, G2, G3, dog, dig, UC, IC, ucent, icent)


def kernel(user_emb, item_emb, user_centroids, item_centroids, edge_index,
           user_2cluster, item_2cluster, user, pos_item, neg_item):
    all0 = jnp.concatenate([user_emb, item_emb], axis=0)
    all0p = jnp.pad(all0, ((0, NP - NN), (0, 0)))
    all0s = all0p.reshape(NP, 2, HD).transpose(1, 0, 2)

    ei = edge_index.astype(i32)
    src3 = ei[0].reshape(NTILES, NCH, W)
    dst3 = ei[1].reshape(NTILES, NCH, W)

    deg = _deg_call(ei)
    deg3 = deg.reshape(2, NP, 1)
    hs0, do_, di_, dido = _prep_call(all0s, deg3)

    agg1 = _layer_call(hs0, src3, dst3)
    hs1 = _scale_call(agg1, dido)
    agg2 = _layer_call(hs1, src3, dst3)
    hs2 = _scale_call(agg2, dido)
    agg3 = _layer_call(hs2, src3, dst3)

    user32 = user.astype(i32)
    pos32 = pos_item.astype(i32)
    ids_all = jnp.concatenate([user32, pos32 + NU,
                               neg_item.astype(i32) + NU])
    G0s, G1s, G2s, G3s, dog, dig, UC, IC = _gather_call(
        all0s, hs1, hs2, agg3, do_, di_, ids_all, user32, pos32,
        user_2cluster.astype(i32), item_2cluster.astype(i32),
        user_centroids, item_centroids)

    G0 = jnp.concatenate([G0s[0], G0s[1]], axis=1)
    G1 = jnp.concatenate([G1s[0], G1s[1]], axis=1)
    G2 = jnp.concatenate([G2s[0], G2s[1]], axis=1)
    G3 = jnp.concatenate([G3s[0], G3s[1]], axis=1)

    out = _loss_call(all0p, G0, G1, G2, G3, dog, dig, UC, IC,
                     user_centroids, item_centroids)
    return out[0, :3]


# speedup vs baseline: 2.8291x; 2.8291x over previous
import functools

import jax
import jax.numpy as jnp
from jax import lax
from jax.experimental import pallas as pl
from jax.experimental.pallas import tpu as pltpu
from jax.experimental.pallas import tpu_sc as plsc

NU = 20000
NI = 30000
NN = NU + NI
NP = 50176
D = 64
HD = 8
NQ = 8
NPASS = NQ // 2
B = 1024
K = 1000
E = 800000
TEMP = 0.1
NTILES = 16
ROWS_PT = NP // NTILES
EDGES_PT = E // NTILES
W = 80
NCH = EDGES_PT // W
ZR = 224
NZ = ROWS_PT // ZR
HRC = 57344
CH = 1568
NSTEP = NP // CH

_MESH = plsc.VectorSubcoreMesh(core_axis_name="c", subcore_axis_name="s")
_SC_PARAMS = pltpu.CompilerParams(use_tc_tiling_on_sc=False)
f32 = jnp.float32
i32 = jnp.int32


NE_PT = HRC // NTILES
DEG_F = 5


def _deg_body(sd3, deg2, idx, ones, zbuf, acc, sem):
    c = lax.axis_index("c")
    s = lax.axis_index("s")

    pltpu.sync_copy(sd3.at[c].at[s], idx)

    zero16 = jnp.zeros((16,), f32)
    for t in range(W // 16):
        ones[pl.ds(t * 16, 16)] = jnp.ones((16,), f32)

    def zb(k, _):
        zbuf[pl.ds(k * 16, 16)] = zero16
        return 0

    lax.fori_loop(0, NE_PT // 16, zb, 0)
    pltpu.sync_copy(zbuf, acc.at[pl.ds(s * NE_PT, NE_PT)])
    plsc.subcore_barrier()

    def outer(j, _):
        for t in range(DEG_F):
            pltpu.async_copy(ones, acc.at[idx.at[j * DEG_F + t]], sem,
                             add=True)
        for t in range(DEG_F):
            pltpu.make_async_copy(ones, acc.at[idx.at[0]], sem).wait()
        return 0

    lax.fori_loop(0, NCH // DEG_F, outer, 0)
    plsc.subcore_barrier()

    pltpu.sync_copy(acc.at[pl.ds(s * NE_PT, NE_PT)],
                    deg2.at[c, pl.ds(s * NE_PT, NE_PT)])


@jax.jit
def _deg_call(sd3):
    return pl.kernel(
        _deg_body,
        out_type=jax.ShapeDtypeStruct((2, HRC), f32),
        mesh=_MESH,
        compiler_params=_SC_PARAMS,
        scratch_types=[
            pltpu.VMEM((NCH, W), i32),
            pltpu.VMEM((W,), f32),
            pltpu.VMEM((NE_PT,), f32),
            pltpu.VMEM_SHARED((HRC,), f32),
            pltpu.SemaphoreType.DMA,
        ],
    )(sd3)


def _layer_body(hs, src3, dst3, zeros_h, agg, srcb, dstb, rows0, rows1,
                semg0, semg1, sems0, sems1, acc):
    c = lax.axis_index("c")
    s = lax.axis_index("s")
    pltpu.sync_copy(src3.at[s], srcb)
    pltpu.sync_copy(dst3.at[s], dstb)

    def wait_g(tbl, buf, sem):
        pltpu.make_async_copy(tbl.at[srcb.at[0]], buf, sem).wait()

    def wait_s(buf, sem):
        pltpu.make_async_copy(buf, acc.at[dstb.at[0]], sem).wait()

    for p in range(NPASS):
        q = 2 * p + c
        tbl = hs.at[q]

        def gather(k, buf, sem, _tbl=tbl):
            pltpu.async_copy(_tbl.at[srcb.at[k]], buf, sem)

        def scat(k, buf, sem):
            pltpu.async_copy(buf, acc.at[dstb.at[k]], sem, add=True)

        pltpu.sync_copy(zeros_h.at[pl.ds(s * ROWS_PT, ROWS_PT)],
                        acc.at[pl.ds(s * ROWS_PT, ROWS_PT)])
        plsc.subcore_barrier()

        gather(0, rows0, semg0)

        def body(j, _, _tbl=tbl, _gather=gather, _scat=scat):
            k0 = 2 * j

            @pl.when(j > 0)
            def _():
                wait_s(rows1, sems1)

            _gather(k0 + 1, rows1, semg1)
            wait_g(_tbl, rows0, semg0)
            _scat(k0, rows0, sems0)

            wait_s(rows0, sems0)

            @pl.when(j < NCH // 2 - 1)
            def _():
                _gather(k0 + 2, rows0, semg0)

            wait_g(_tbl, rows1, semg1)
            _scat(k0 + 1, rows1, sems1)
            return 0

        lax.fori_loop(0, NCH // 2, body, 0)
        wait_s(rows1, sems1)
        gather(NCH - 1, rows0, semg0)
        wait_g(tbl, rows0, semg0)
        scat(NCH - 1, rows0, sems0)
        wait_s(rows0, sems0)
        plsc.subcore_barrier()

        for t in range(NZ):
            r0 = s * ROWS_PT + t * ZR
            pltpu.sync_copy(acc.at[pl.ds(r0, ZR)], agg.at[q, pl.ds(r0, ZR)])


@jax.jit
def _layer_call(hs, src3, dst3, zeros_h):
    return pl.kernel(
        _layer_body,
        out_type=jax.ShapeDtypeStruct((NQ, NP, HD), f32),
        mesh=_MESH,
        compiler_params=_SC_PARAMS,
        scratch_types=[
            pltpu.VMEM((NCH, W), i32),
            pltpu.VMEM((NCH, W), i32),
            pltpu.VMEM((W, HD), f32),
            pltpu.VMEM((W, HD), f32),
            pltpu.SemaphoreType.DMA,
            pltpu.SemaphoreType.DMA,
            pltpu.SemaphoreType.DMA,
            pltpu.SemaphoreType.DMA,
            pltpu.VMEM_SHARED((NP, HD), f32),
        ],
    )(hs, src3, dst3, zeros_h)


def _scalemul_body(tblf, sca, outf, rows_v, dsc_v):
    c = lax.axis_index("c")
    s = lax.axis_index("s")
    pltpu.sync_copy(sca.at[pl.ds(s * ROWS_PT, ROWS_PT)],
                    dsc_v.at[pl.ds(0, ROWS_PT)])
    lane16 = lax.iota(i32, 16)

    for dq in range(NPASS):
        q = c * NPASS + dq
        for t in range(NZ):
            off = (s * ROWS_PT + t * ZR) * HD
            pltpu.sync_copy(tblf.at[q, pl.ds(off, ZR * HD)], rows_v)

            def sb(k, _, _t=t):
                v = rows_v[pl.ds(k * 16, 16)]
                dv = dsc_v[pl.ds(_t * ZR + 2 * k, 16)]
                scv = jnp.where(lane16 < HD, dv[0], dv[1])
                rows_v[pl.ds(k * 16, 16)] = v * scv
                return 0

            lax.fori_loop(0, ZR * HD // 16, sb, 0)
            pltpu.sync_copy(rows_v, outf.at[q, pl.ds(off, ZR * HD)])


@jax.jit
def _scalemul_call(tblf, sca):
    return pl.kernel(
        _scalemul_body,
        out_type=jax.ShapeDtypeStruct((NQ, NP * HD), f32),
        mesh=_MESH,
        compiler_params=_SC_PARAMS,
        scratch_types=[
            pltpu.VMEM((ZR * HD,), f32),
            pltpu.VMEM((ROWS_PT + 16,), f32),
        ],
    )(tblf, sca)


def _gather_body(all0s, hs1, hs2, agg3, dodi_h, ids_h, up2, cc2, cent2,
                 G0, G1, G2, G3, dd, UCIC,
                 idsb, rowsb, svalb, sidb, clb, crows, sem):
    c = lax.axis_index("c")
    s = lax.axis_index("s")

    for j in range(2):
        base = s * 192 + j * 96
        pltpu.sync_copy(ids_h.at[pl.ds(base, 96)], idsb)
        for tbl, out in ((all0s, G0), (hs1, G1), (hs2, G2), (agg3, G3)):
            for dq in range(NPASS):
                q = 2 * dq + c
                pltpu.async_copy(tbl.at[q].at[idsb], rowsb, sem).wait()
                pltpu.sync_copy(rowsb, out.at[q, pl.ds(base, 96)])
        pltpu.async_copy(dodi_h.at[c].at[idsb], svalb, sem).wait()
        pltpu.sync_copy(svalb, dd.at[c, pl.ds(base, 96)])

    cb = s * 64
    pltpu.sync_copy(up2.at[c, pl.ds(cb, 64)], sidb)
    pltpu.async_copy(cc2.at[c].at[sidb], clb, sem).wait()
    pltpu.async_copy(cent2.at[c].at[clb], crows, sem).wait()
    pltpu.sync_copy(crows, UCIC.at[c, pl.ds(cb, 64)])


@jax.jit
def _gather_call(all0s, hs1, hs2, agg3, dodi_h, ids_all, up2, cc2, cent2):
    return pl.kernel(
        _gather_body,
        out_type=(
            jax.ShapeDtypeStruct((NQ, 3 * B, HD), f32),
            jax.ShapeDtypeStruct((NQ, 3 * B, HD), f32),
            jax.ShapeDtypeStruct((NQ, 3 * B, HD), f32),
            jax.ShapeDtypeStruct((NQ, 3 * B, HD), f32),
            jax.ShapeDtypeStruct((2, 3 * B, 1), f32),
            jax.ShapeDtypeStruct((2, B, D), f32),
        ),
        mesh=_MESH,
        compiler_params=_SC_PARAMS,
        scratch_types=[
            pltpu.VMEM((96,), i32),
            pltpu.VMEM((96, HD), f32),
            pltpu.VMEM((96, 1), f32),
            pltpu.VMEM((64,), i32),
            pltpu.VMEM((64,), i32),
            pltpu.VMEM((64, D), f32),
            pltpu.SemaphoreType.DMA,
        ],
    )(all0s, hs1, hs2, agg3, dodi_h, ids_all, up2, cc2, cent2)


RB = 6272


def _prep_body(deg_ref, dodi_ref, dido_ref):
    dego = deg_ref[0]
    degi = deg_ref[1]
    do = lax.rsqrt(jnp.where(dego > 0, dego, 1.0))
    di = lax.rsqrt(jnp.where(degi > 0, degi, 1.0))
    dodi_ref[0] = do
    dodi_ref[1] = di
    dido_ref[...] = do * di


@jax.jit
def _prep_call(deg3):
    return pl.pallas_call(
        _prep_body,
        grid=(NP // RB,),
        in_specs=[pl.BlockSpec((2, RB, 1), lambda i: (0, i, 0))],
        out_specs=[
            pl.BlockSpec((2, RB, 1), lambda i: (0, i, 0)),
            pl.BlockSpec((RB, 1), lambda i: (i, 0)),
        ],
        out_shape=[
            jax.ShapeDtypeStruct((2, NP, 1), f32),
            jax.ShapeDtypeStruct((NP, 1), f32),
        ],
    )(deg3)


def _loss_body(all0p_ref, G0r, G1r, G2r, G3r, dogr, digr, UCr, ICr,
               ucr, icr, out_ref, n1_s, ttl_s):
    i = pl.program_id(0)
    invT = 1.0 / TEMP

    def l2n(v):
        nrm = jnp.sqrt(jnp.sum(v * v, axis=1, keepdims=True))
        return v / jnp.maximum(nrm, 1e-12)

    @pl.when(i == 0)
    def _():
        g2 = G2r[...]
        n1_s[...] = l2n(g2[0:2 * B])
        ttl_s[...] = jnp.zeros((2 * B, 1), f32)

    x = all0p_ref[...]
    xn = l2n(x)
    lg = lax.dot_general(n1_s[...], xn, (((1,), (1,)), ((), ())),
                         preferred_element_type=f32)
    col = i * CH + lax.broadcasted_iota(i32, (1, CH), 1)
    row = lax.broadcasted_iota(i32, (2 * B, 1), 0)
    rowu = (row < B).astype(f32)
    mcu = (col < NU).astype(f32)
    mci = ((col >= NU) & (col < NN)).astype(f32)
    m = rowu * mcu + (1.0 - rowu) * mci
    e = jnp.exp(lg * invT) * m
    ttl_s[...] += jnp.sum(e, axis=1, keepdims=True)

    @pl.when(i == NSTEP - 1)
    def _():
        g0 = G0r[...]
        inv_do = 1.0 / dogr[...]
        h1 = G1r[...] * inv_do
        h2 = G2r[...] * inv_do
        h3 = G3r[...] * digr[...]
        lgall = (g0 + h1 + h2 + h3) * 0.25
        ue = lgall[0:B]
        pe = lgall[B:2 * B]
        ne = lgall[2 * B:3 * B]
        pos_s = jnp.sum(ue * pe, axis=1)
        neg_s = jnp.sum(ue * ne, axis=1)
        sig = 1.0 / (1.0 + jnp.exp(neg_s - pos_s))
        mf = -jnp.mean(jnp.log(1e-10 + sig))
        g0u = g0[0:B]
        g0p = g0[B:2 * B]
        g0n = g0[2 * B:3 * B]
        reg = (jnp.sqrt(jnp.sum(g0u * g0u)) + jnp.sqrt(jnp.sum(g0p * g0p))
               + jnp.sqrt(jnp.sum(g0n * g0n))) / B
        n1 = n1_s[...]
        nu2 = l2n(g0u)
        ni2 = l2n(g0p)
        n2 = jnp.concatenate([nu2, ni2], axis=0)
        pos_ = jnp.exp(jnp.sum(n1 * n2, axis=1, keepdims=True) * invT)
        ssl_total = -jnp.sum(jnp.log(pos_ / ttl_s[...]))
        uc = UCr[...]
        ic = ICr[...]
        pos_su = jnp.exp(jnp.sum(nu2 * uc, axis=1) * invT)
        lsu = lax.dot_general(nu2, ucr[...], (((1,), (1,)), ((), ())),
                              preferred_element_type=f32)
        ttl_su = jnp.sum(jnp.exp(lsu * invT), axis=1)
        proto_u = -jnp.sum(jnp.log(pos_su / ttl_su))
        pos_si = jnp.exp(jnp.sum(ni2 * ic, axis=1) * invT)
        lsi = lax.dot_general(ni2, icr[...], (((1,), (1,)), ((), ())),
                              preferred_element_type=f32)
        ttl_si = jnp.sum(jnp.exp(lsi * invT), axis=1)
        proto_i = -jnp.sum(jnp.log(pos_si / ttl_si))
        l0 = mf + 1e-4 * reg
        l1 = 1e-6 * ssl_total
        l2v = 8e-8 * (proto_u + proto_i)
        lane = lax.broadcasted_iota(i32, (1, 128), 1)
        out_ref[...] = jnp.where(
            lane == 0, l0, jnp.where(lane == 1, l1,
                                     jnp.where(lane == 2, l2v, 0.0)))


@jax.jit
def _loss_call(all0p, G0, G1, G2, G3, dog, dig, UC, IC, ucent, icent):
    def full(shape):
        return pl.BlockSpec(shape, lambda i, _s=shape: tuple(0 for _ in _s))
    return pl.pallas_call(
        _loss_body,
        grid=(NSTEP,),
        in_specs=[
            pl.BlockSpec((CH, D), lambda i: (i, 0)),
            full((3 * B, D)), full((3 * B, D)), full((3 * B, D)),
            full((3 * B, D)), full((3 * B, 1)), full((3 * B, 1)),
            full((B, D)), full((B, D)), full((K, D)), full((K, D)),
        ],
        out_specs=pl.BlockSpec((1, 128), lambda i: (0, 0)),
        out_shape=jax.ShapeDtypeStruct((1, 128), f32),
        scratch_shapes=[
            pltpu.VMEM((2 * B, D), f32),
            pltpu.VMEM((2 * B, 1), f32),
        ],
    )(all0p, G0, G1, G2, G3, dog, dig, UC, IC, ucent, icent)


def kernel(user_emb, item_emb, user_centroids, item_centroids, edge_index,
           user_2cluster, item_2cluster, user, pos_item, neg_item):
    all0 = jnp.concatenate([user_emb, item_emb], axis=0)
    all0p = jnp.pad(all0, ((0, NP - NN), (0, 0)))
    all0s = all0p.reshape(NP, NQ, HD).transpose(1, 0, 2)

    ei = edge_index.astype(i32)
    src3 = ei[0].reshape(NTILES, NCH, W)
    dst3 = ei[1].reshape(NTILES, NCH, W)
    zeros_h = jnp.zeros((NP, HD), f32)

    deg2 = _deg_call(ei.reshape(2, NTILES, NCH, W))
    deg3 = deg2[:, :NP].reshape(2, NP, 1)
    dodi, dido = _prep_call(deg3)
    dof = dodi[0].reshape(NP)
    didof = dido.reshape(NP)

    def scale(tbl, sca):
        return _scalemul_call(tbl.reshape(NQ, NP * HD),
                              sca).reshape(NQ, NP, HD)

    def layer(hs):
        return _layer_call(hs, src3, dst3, zeros_h)

    hs0 = scale(all0s, dof)
    agg1 = layer(hs0)
    hs1 = scale(agg1, didof)
    agg2 = layer(hs1)
    hs2 = scale(agg2, didof)
    agg3 = layer(hs2)

    user32 = user.astype(i32)
    pos32 = pos_item.astype(i32)
    ids_all = jnp.concatenate([user32, pos32 + NU,
                               neg_item.astype(i32) + NU])
    up2 = jnp.stack([user32, pos32], axis=0)
    cc2 = jnp.stack([jnp.pad(user_2cluster.astype(i32), (0, NI - NU)),
                     item_2cluster.astype(i32)], axis=0)
    cent2 = jnp.stack([user_centroids, item_centroids], axis=0)
    G0s, G1s, G2s, G3s, dd, UCIC = _gather_call(
        all0s, hs1, hs2, agg3, dodi, ids_all, up2, cc2, cent2)
    dog = dd[0]
    dig = dd[1]
    UC = UCIC[0]
    IC = UCIC[1]

    def asm(Gq):
        return jnp.concatenate([Gq[q] for q in range(NQ)], axis=1)

    G0 = asm(G0s)
    G1 = asm(G1s)
    G2 = asm(G2s)
    G3 = asm(G3s)

    out = _loss_call(all0p, G0, G1, G2, G3, dog, dig, UC, IC,
                     user_centroids, item_centroids)
    return out[0, :3]


# speedup vs baseline: 3.5983x; 1.2719x over previous
import functools

import jax
import jax.numpy as jnp
from jax import lax
from jax.experimental import pallas as pl
from jax.experimental.pallas import tpu as pltpu
from jax.experimental.pallas import tpu_sc as plsc

NU = 20000
NI = 30000
NN = NU + NI
NP = 50176
D = 64
HD = 8
NQ = 8
NPASS = NQ // 2
B = 1024
K = 1000
E = 800000
TEMP = 0.1
NTILES = 16
ROWS_PT = NP // NTILES
EDGES_PT = E // NTILES
W = 128
EP_PAD = 50048
E_PAD = EP_PAD * NTILES
NCH = EP_PAD // W
ZR = 224
NZ = ROWS_PT // ZR
HRC = 57344
CH = 1568
NSTEP = NP // CH

_MESH = plsc.VectorSubcoreMesh(core_axis_name="c", subcore_axis_name="s")
_SC_PARAMS = pltpu.CompilerParams(use_tc_tiling_on_sc=False)
f32 = jnp.float32
i32 = jnp.int32


NE_PT = HRC // NTILES
DEG_F = 17


def _deg_body(sd3, deg2, idx, ones, zbuf, acc, sem):
    c = lax.axis_index("c")
    s = lax.axis_index("s")

    pltpu.sync_copy(sd3.at[c].at[s], idx)

    zero16 = jnp.zeros((16,), f32)
    for t in range(W // 16):
        ones[pl.ds(t * 16, 16)] = jnp.ones((16,), f32)

    def zb(k, _):
        zbuf[pl.ds(k * 16, 16)] = zero16
        return 0

    lax.fori_loop(0, NE_PT // 16, zb, 0)
    pltpu.sync_copy(zbuf, acc.at[pl.ds(s * NE_PT, NE_PT)])
    plsc.subcore_barrier()

    def outer(j, _):
        for t in range(DEG_F):
            pltpu.async_copy(ones, acc.at[idx.at[j * DEG_F + t]], sem,
                             add=True)
        for t in range(DEG_F):
            pltpu.make_async_copy(ones, acc.at[idx.at[0]], sem).wait()
        return 0

    lax.fori_loop(0, NCH // DEG_F, outer, 0)
    plsc.subcore_barrier()

    pltpu.sync_copy(acc.at[pl.ds(s * NE_PT, NE_PT)],
                    deg2.at[c, pl.ds(s * NE_PT, NE_PT)])


@jax.jit
def _deg_call(sd3):
    return pl.kernel(
        _deg_body,
        out_type=jax.ShapeDtypeStruct((2, HRC), f32),
        mesh=_MESH,
        compiler_params=_SC_PARAMS,
        scratch_types=[
            pltpu.VMEM((NCH, W), i32),
            pltpu.VMEM((W,), f32),
            pltpu.VMEM((NE_PT,), f32),
            pltpu.VMEM_SHARED((HRC,), f32),
            pltpu.SemaphoreType.DMA,
        ],
    )(sd3)


def _layer_body(hs, src3, dst3, zeros_h, agg, srcb, dstb, rows0, rows1,
                semg0, semg1, sems0, sems1, acc):
    c = lax.axis_index("c")
    s = lax.axis_index("s")
    pltpu.sync_copy(src3.at[s], srcb)
    pltpu.sync_copy(dst3.at[s], dstb)

    def wait_g(tbl, buf, sem):
        pltpu.make_async_copy(tbl.at[srcb.at[0]], buf, sem).wait()

    def wait_s(buf, sem):
        pltpu.make_async_copy(buf, acc.at[dstb.at[0]], sem).wait()

    for p in range(NPASS):
        q = 2 * p + c
        tbl = hs.at[q]

        def gather(k, buf, sem, _tbl=tbl):
            pltpu.async_copy(_tbl.at[srcb.at[k]], buf, sem)

        def scat(k, buf, sem):
            pltpu.async_copy(buf, acc.at[dstb.at[k]], sem, add=True)

        pltpu.sync_copy(zeros_h.at[pl.ds(s * ROWS_PT, ROWS_PT)],
                        acc.at[pl.ds(s * ROWS_PT, ROWS_PT)])
        plsc.subcore_barrier()

        gather(0, rows0, semg0)

        def body(j, _, _tbl=tbl, _gather=gather, _scat=scat):
            k0 = 2 * j

            @pl.when(j > 0)
            def _():
                wait_s(rows1, sems1)

            _gather(k0 + 1, rows1, semg1)
            wait_g(_tbl, rows0, semg0)
            _scat(k0, rows0, sems0)

            wait_s(rows0, sems0)

            @pl.when(j < NCH // 2 - 1)
            def _():
                _gather(k0 + 2, rows0, semg0)

            wait_g(_tbl, rows1, semg1)
            _scat(k0 + 1, rows1, sems1)
            return 0

        lax.fori_loop(0, NCH // 2, body, 0)
        wait_s(rows1, sems1)
        gather(NCH - 1, rows0, semg0)
        wait_g(tbl, rows0, semg0)
        scat(NCH - 1, rows0, sems0)
        wait_s(rows0, sems0)
        plsc.subcore_barrier()

        for t in range(NZ):
            r0 = s * ROWS_PT + t * ZR
            pltpu.sync_copy(acc.at[pl.ds(r0, ZR)], agg.at[q, pl.ds(r0, ZR)])


@jax.jit
def _layer_call(hs, src3, dst3, zeros_h):
    return pl.kernel(
        _layer_body,
        out_type=jax.ShapeDtypeStruct((NQ, NP, HD), f32),
        mesh=_MESH,
        compiler_params=_SC_PARAMS,
        scratch_types=[
            pltpu.VMEM((NCH, W), i32),
            pltpu.VMEM((NCH, W), i32),
            pltpu.VMEM((W, HD), f32),
            pltpu.VMEM((W, HD), f32),
            pltpu.SemaphoreType.DMA,
            pltpu.SemaphoreType.DMA,
            pltpu.SemaphoreType.DMA,
            pltpu.SemaphoreType.DMA,
            pltpu.VMEM_SHARED((NP, HD), f32),
        ],
    )(hs, src3, dst3, zeros_h)


def _scalemul_body(tblf, sca, outf, rows_v, dsc_v):
    c = lax.axis_index("c")
    s = lax.axis_index("s")
    pltpu.sync_copy(sca.at[pl.ds(s * ROWS_PT, ROWS_PT)],
                    dsc_v.at[pl.ds(0, ROWS_PT)])
    lane16 = lax.iota(i32, 16)

    for dq in range(NPASS):
        q = c * NPASS + dq
        for t in range(NZ):
            off = (s * ROWS_PT + t * ZR) * HD
            pltpu.sync_copy(tblf.at[q, pl.ds(off, ZR * HD)], rows_v)

            def sb(k, _, _t=t):
                v = rows_v[pl.ds(k * 16, 16)]
                dv = dsc_v[pl.ds(_t * ZR + 2 * k, 16)]
                scv = jnp.where(lane16 < HD, dv[0], dv[1])
                rows_v[pl.ds(k * 16, 16)] = v * scv
                return 0

            lax.fori_loop(0, ZR * HD // 16, sb, 0)
            pltpu.sync_copy(rows_v, outf.at[q, pl.ds(off, ZR * HD)])


@jax.jit
def _scalemul_call(tblf, sca):
    return pl.kernel(
        _scalemul_body,
        out_type=jax.ShapeDtypeStruct((NQ, NP * HD), f32),
        mesh=_MESH,
        compiler_params=_SC_PARAMS,
        scratch_types=[
            pltpu.VMEM((ZR * HD,), f32),
            pltpu.VMEM((ROWS_PT + 16,), f32),
        ],
    )(tblf, sca)


def _gather_body(all0s, hs1, hs2, agg3, dodi_h, ids_h, up2, cc2, cent2,
                 G0, G1, G2, G3, dd, UCIC,
                 idsb, rowsb, svalb, sidb, clb, crows, sem):
    c = lax.axis_index("c")
    s = lax.axis_index("s")

    for j in range(2):
        base = s * 192 + j * 96
        pltpu.sync_copy(ids_h.at[pl.ds(base, 96)], idsb)
        for tbl, out in ((all0s, G0), (hs1, G1), (hs2, G2), (agg3, G3)):
            for dq in range(NPASS):
                q = 2 * dq + c
                pltpu.async_copy(tbl.at[q].at[idsb], rowsb, sem).wait()
                pltpu.sync_copy(rowsb, out.at[q, pl.ds(base, 96)])
        pltpu.async_copy(dodi_h.at[c].at[idsb], svalb, sem).wait()
        pltpu.sync_copy(svalb, dd.at[c, pl.ds(base, 96)])

    cb = s * 64
    pltpu.sync_copy(up2.at[c, pl.ds(cb, 64)], sidb)
    pltpu.async_copy(cc2.at[c].at[sidb], clb, sem).wait()
    pltpu.async_copy(cent2.at[c].at[clb], crows, sem).wait()
    pltpu.sync_copy(crows, UCIC.at[c, pl.ds(cb, 64)])


@jax.jit
def _gather_call(all0s, hs1, hs2, agg3, dodi_h, ids_all, up2, cc2, cent2):
    return pl.kernel(
        _gather_body,
        out_type=(
            jax.ShapeDtypeStruct((NQ, 3 * B, HD), f32),
            jax.ShapeDtypeStruct((NQ, 3 * B, HD), f32),
            jax.ShapeDtypeStruct((NQ, 3 * B, HD), f32),
            jax.ShapeDtypeStruct((NQ, 3 * B, HD), f32),
            jax.ShapeDtypeStruct((2, 3 * B, 1), f32),
            jax.ShapeDtypeStruct((2, B, D), f32),
        ),
        mesh=_MESH,
        compiler_params=_SC_PARAMS,
        scratch_types=[
            pltpu.VMEM((96,), i32),
            pltpu.VMEM((96, HD), f32),
            pltpu.VMEM((96, 1), f32),
            pltpu.VMEM((64,), i32),
            pltpu.VMEM((64,), i32),
            pltpu.VMEM((64, D), f32),
            pltpu.SemaphoreType.DMA,
        ],
    )(all0s, hs1, hs2, agg3, dodi_h, ids_all, up2, cc2, cent2)


RB = 6272


def _prep_body(deg_ref, dodi_ref, dido_ref):
    dego = deg_ref[0]
    degi = deg_ref[1]
    do = lax.rsqrt(jnp.where(dego > 0, dego, 1.0))
    di = lax.rsqrt(jnp.where(degi > 0, degi, 1.0))
    dodi_ref[0] = do
    dodi_ref[1] = di
    dido_ref[...] = do * di


@jax.jit
def _prep_call(deg3):
    return pl.pallas_call(
        _prep_body,
        grid=(NP // RB,),
        in_specs=[pl.BlockSpec((2, RB, 1), lambda i: (0, i, 0))],
        out_specs=[
            pl.BlockSpec((2, RB, 1), lambda i: (0, i, 0)),
            pl.BlockSpec((RB, 1), lambda i: (i, 0)),
        ],
        out_shape=[
            jax.ShapeDtypeStruct((2, NP, 1), f32),
            jax.ShapeDtypeStruct((NP, 1), f32),
        ],
    )(deg3)


def _loss_body(all0p_ref, G0r, G1r, G2r, G3r, dogr, digr, UCr, ICr,
               ucr, icr, out_ref, n1_s, ttl_s):
    i = pl.program_id(0)
    invT = 1.0 / TEMP

    def l2n(v):
        nrm = jnp.sqrt(jnp.sum(v * v, axis=1, keepdims=True))
        return v / jnp.maximum(nrm, 1e-12)

    @pl.when(i == 0)
    def _():
        g2 = G2r[...]
        n1_s[...] = l2n(g2[0:2 * B])
        ttl_s[...] = jnp.zeros((2 * B, 1), f32)

    x = all0p_ref[...]
    xn = l2n(x)
    lg = lax.dot_general(n1_s[...], xn, (((1,), (1,)), ((), ())),
                         preferred_element_type=f32)
    col = i * CH + lax.broadcasted_iota(i32, (1, CH), 1)
    row = lax.broadcasted_iota(i32, (2 * B, 1), 0)
    rowu = (row < B).astype(f32)
    mcu = (col < NU).astype(f32)
    mci = ((col >= NU) & (col < NN)).astype(f32)
    m = rowu * mcu + (1.0 - rowu) * mci
    e = jnp.exp(lg * invT) * m
    ttl_s[...] += jnp.sum(e, axis=1, keepdims=True)

    @pl.when(i == NSTEP - 1)
    def _():
        g0 = G0r[...]
        inv_do = 1.0 / dogr[...]
        h1 = G1r[...] * inv_do
        h2 = G2r[...] * inv_do
        h3 = G3r[...] * digr[...]
        lgall = (g0 + h1 + h2 + h3) * 0.25
        ue = lgall[0:B]
        pe = lgall[B:2 * B]
        ne = lgall[2 * B:3 * B]
        pos_s = jnp.sum(ue * pe, axis=1)
        neg_s = jnp.sum(ue * ne, axis=1)
        sig = 1.0 / (1.0 + jnp.exp(neg_s - pos_s))
        mf = -jnp.mean(jnp.log(1e-10 + sig))
        g0u = g0[0:B]
        g0p = g0[B:2 * B]
        g0n = g0[2 * B:3 * B]
        reg = (jnp.sqrt(jnp.sum(g0u * g0u)) + jnp.sqrt(jnp.sum(g0p * g0p))
               + jnp.sqrt(jnp.sum(g0n * g0n))) / B
        n1 = n1_s[...]
        nu2 = l2n(g0u)
        ni2 = l2n(g0p)
        n2 = jnp.concatenate([nu2, ni2], axis=0)
        pos_ = jnp.exp(jnp.sum(n1 * n2, axis=1, keepdims=True) * invT)
        ssl_total = -jnp.sum(jnp.log(pos_ / ttl_s[...]))
        uc = UCr[...]
        ic = ICr[...]
        pos_su = jnp.exp(jnp.sum(nu2 * uc, axis=1) * invT)
        lsu = lax.dot_general(nu2, ucr[...], (((1,), (1,)), ((), ())),
                              preferred_element_type=f32)
        ttl_su = jnp.sum(jnp.exp(lsu * invT), axis=1)
        proto_u = -jnp.sum(jnp.log(pos_su / ttl_su))
        pos_si = jnp.exp(jnp.sum(ni2 * ic, axis=1) * invT)
        lsi = lax.dot_general(ni2, icr[...], (((1,), (1,)), ((), ())),
                              preferred_element_type=f32)
        ttl_si = jnp.sum(jnp.exp(lsi * invT), axis=1)
        proto_i = -jnp.sum(jnp.log(pos_si / ttl_si))
        l0 = mf + 1e-4 * reg
        l1 = 1e-6 * ssl_total
        l2v = 8e-8 * (proto_u + proto_i)
        lane = lax.broadcasted_iota(i32, (1, 128), 1)
        out_ref[...] = jnp.where(
            lane == 0, l0, jnp.where(lane == 1, l1,
                                     jnp.where(lane == 2, l2v, 0.0)))


@jax.jit
def _loss_call(all0p, G0, G1, G2, G3, dog, dig, UC, IC, ucent, icent):
    def full(shape):
        return pl.BlockSpec(shape, lambda i, _s=shape: tuple(0 for _ in _s))
    return pl.pallas_call(
        _loss_body,
        grid=(NSTEP,),
        in_specs=[
            pl.BlockSpec((CH, D), lambda i: (i, 0)),
            full((3 * B, D)), full((3 * B, D)), full((3 * B, D)),
            full((3 * B, D)), full((3 * B, 1)), full((3 * B, 1)),
            full((B, D)), full((B, D)), full((K, D)), full((K, D)),
        ],
        out_specs=pl.BlockSpec((1, 128), lambda i: (0, 0)),
        out_shape=jax.ShapeDtypeStruct((1, 128), f32),
        scratch_shapes=[
            pltpu.VMEM((2 * B, D), f32),
            pltpu.VMEM((2 * B, 1), f32),
        ],
    )(all0p, G0, G1, G2, G3, dog, dig, UC, IC, ucent, icent)


def kernel(user_emb, item_emb, user_centroids, item_centroids, edge_index,
           user_2cluster, item_2cluster, user, pos_item, neg_item):
    all0 = jnp.concatenate([user_emb, item_emb], axis=0)
    all0p = jnp.pad(all0, ((0, NP - NN), (0, 0)))
    all0s = all0p.reshape(NP, NQ, HD).transpose(1, 0, 2)

    ei = edge_index.astype(i32)
    pad = NN + (jnp.arange(E_PAD - E, dtype=i32) % (NP - NN))
    srcp = jnp.concatenate([ei[0], pad])
    dstp = jnp.concatenate([ei[1], pad])
    src3 = srcp.reshape(NTILES, NCH, W)
    dst3 = dstp.reshape(NTILES, NCH, W)
    zeros_h = jnp.zeros((NP, HD), f32)

    deg2 = _deg_call(jnp.stack([src3, dst3], axis=0))
    deg3 = deg2[:, :NP].reshape(2, NP, 1)
    dodi, dido = _prep_call(deg3)
    dof = dodi[0].reshape(NP)
    didof = dido.reshape(NP)

    def scale(tbl, sca):
        return _scalemul_call(tbl.reshape(NQ, NP * HD),
                              sca).reshape(NQ, NP, HD)

    def layer(hs):
        return _layer_call(hs, src3, dst3, zeros_h)

    hs0 = scale(all0s, dof)
    agg1 = layer(hs0)
    hs1 = scale(agg1, didof)
    agg2 = layer(hs1)
    hs2 = scale(agg2, didof)
    agg3 = layer(hs2)

    user32 = user.astype(i32)
    pos32 = pos_item.astype(i32)
    ids_all = jnp.concatenate([user32, pos32 + NU,
                               neg_item.astype(i32) + NU])
    up2 = jnp.stack([user32, pos32], axis=0)
    cc2 = jnp.stack([jnp.pad(user_2cluster.astype(i32), (0, NI - NU)),
                     item_2cluster.astype(i32)], axis=0)
    cent2 = jnp.stack([user_centroids, item_centroids], axis=0)
    G0s, G1s, G2s, G3s, dd, UCIC = _gather_call(
        all0s, hs1, hs2, agg3, dodi, ids_all, up2, cc2, cent2)
    dog = dd[0]
    dig = dd[1]
    UC = UCIC[0]
    IC = UCIC[1]

    def asm(Gq):
        return jnp.concatenate([Gq[q] for q in range(NQ)], axis=1)

    G0 = asm(G0s)
    G1 = asm(G1s)
    G2 = asm(G2s)
    G3 = asm(G3s)

    out = _loss_call(all0p, G0, G1, G2, G3, dog, dig, UC, IC,
                     user_centroids, item_centroids)
    return out[0, :3]
